# baseline (speedup 1.0000x reference)
import functools

import jax
import jax.numpy as jnp
from jax import lax
from jax.experimental import pallas as pl
from jax.experimental.pallas import tpu as pltpu
from jax.experimental.pallas import tpu_sc as plsc

_VOCAB = 1000000
_DIM = 64
_BATCH = 16384
_TOTAL = 3 * _BATCH

_NC = 2
_NS = 16
_NW = _NC * _NS
_HPW = _TOTAL // _NW

_LBLK = 896
_NFULL = _VOCAB // _LBLK
_TAIL = _VOCAB - _NFULL * _LBLK

_RING = 16

_mesh = plsc.VectorSubcoreMesh(core_axis_name="c", subcore_axis_name="s")


@functools.partial(
    pl.kernel,
    out_type=jax.ShapeDtypeStruct((_TOTAL * _DIM,), jnp.float32),
    mesh=_mesh,
    compiler_params=pltpu.CompilerParams(
        use_tc_tiling_on_sc=True, needs_layout_passes=False
    ),
    scratch_types=[
        pltpu.VMEM((_HPW // 128, 128), jnp.int32),
        pltpu.VMEM((_HPW // 128, 128), jnp.int32),
        pltpu.VMEM((2, _DIM, _LBLK), jnp.float32),
        pltpu.VMEM((_RING, _DIM), jnp.float32),
        pltpu.SemaphoreType.DMA,
        pltpu.SemaphoreType.DMA,
    ],
)
def _gather_kernel(tableT, tail_pad, ids_hbm, pos_hbm, out_hbm, ids_v, pos_v,
                   bufs_v, ring_v, sem_out, sem_blk):
    wid = lax.axis_index("s") * _NC + lax.axis_index("c")
    pltpu.sync_copy(ids_hbm.at[wid], ids_v)
    pltpu.sync_copy(pos_hbm.at[wid], pos_v)

    first_id = ids_v[0, pl.ds(0, 16)][0]
    lane = lax.iota(jnp.int32, 16)

    def drain_one_row():
        pltpu.make_async_copy(
            out_hbm.at[pl.ds(0, _DIM)], ring_v.at[0], sem_out
        ).wait()

    def wait_block():
        pltpu.make_async_copy(
            tableT.at[:, pl.ds(0, _LBLK)], bufs_v.at[0], sem_blk
        ).wait()

    def prefetch_block(pred, b, buf_slot):
        @pl.when(jnp.logical_and(pred, b != _NFULL))
        def _():
            pltpu.async_copy(
                tableT.at[:, pl.ds(b * _LBLK, _LBLK)],
                bufs_v.at[buf_slot], sem_blk,
            )

        @pl.when(jnp.logical_and(pred, b == _NFULL))
        def _():
            pltpu.async_copy(tail_pad, bufs_v.at[buf_slot], sem_blk)

    def process_group(g, carry):
        cur_blk, parity, pid = carry
        row = jnp.full((16,), g // 8, jnp.int32)
        colg = (lax.rem(g, jnp.int32(8)) * 16) + lane
        ids16 = plsc.load_gather(ids_v, [row, colg])
        pos16 = plsc.load_gather(pos_v, [row, colg])

        for j in range(16):
            nid = ids16[j]
            npos = pos16[j]
            blk = nid // _LBLK
            switch = blk != cur_blk
            hit_pf = jnp.logical_and(switch, pid == blk)
            gap = jnp.logical_and(switch, pid != blk)

            @pl.when(switch)
            def _():
                wait_block()

            @pl.when(jnp.logical_and(gap, blk != _NFULL))
            def _():
                pltpu.sync_copy(
                    tableT.at[:, pl.ds(blk * _LBLK, _LBLK)],
                    bufs_v.at[parity],
                )

            @pl.when(jnp.logical_and(gap, blk == _NFULL))
            def _():
                pltpu.sync_copy(tail_pad, bufs_v.at[parity])

            parity = jnp.where(hit_pf, 1 - parity, parity)
            cur_blk = jnp.where(switch, blk, cur_blk)
            tp = jnp.minimum(blk + 1, _NFULL)
            prefetch_block(switch, tp, 1 - parity)
            pid = jnp.where(switch, tp, pid)

            c = nid - blk * _LBLK
            slot = j

            drain_one_row()

            par16 = jnp.full((16,), parity, jnp.int32)
            col = jnp.full((16,), c, jnp.int32)
            for q in range(_DIM // 16):
                v = plsc.load_gather(bufs_v, [par16, lane + (16 * q), col])
                ring_v[slot, pl.ds(16 * q, 16)] = v
            pltpu.async_copy(
                ring_v.at[slot], out_hbm.at[pl.ds(npos * _DIM, _DIM)],
                sem_out,
            )

        return (cur_blk, parity, pid)

    for s in range(_RING):
        pltpu.async_copy(out_hbm.at[pl.ds(0, _DIM)], ring_v.at[s], sem_out)

    b0 = first_id // _LBLK

    @pl.when(b0 != _NFULL)
    def _():
        pltpu.sync_copy(tableT.at[:, pl.ds(b0 * _LBLK, _LBLK)], bufs_v.at[0])

    @pl.when(b0 == _NFULL)
    def _():
        pltpu.sync_copy(tail_pad, bufs_v.at[0])

    tp0 = jnp.minimum(b0 + 1, _NFULL)
    prefetch_block(jnp.bool_(True), tp0, 1)

    carry0 = (b0, jnp.int32(0), tp0)
    lax.fori_loop(0, _HPW // 16, process_group, carry0)

    wait_block()
    for _ in range(_RING):
        drain_one_row()


def kernel(anchor, pos, neg, embedding_table):
    tableT = embedding_table.T
    tail_pad = jnp.zeros((_DIM, _LBLK), jnp.float32)
    tail_pad = tail_pad.at[:, :_TAIL].set(embedding_table[_NFULL * _LBLK:, :].T)
    idx = jnp.concatenate([anchor, pos, neg]).astype(jnp.int32)
    sorted_ids, order = lax.sort_key_val(
        idx, lax.iota(jnp.int32, _TOTAL)
    )
    out = _gather_kernel(
        tableT,
        tail_pad,
        sorted_ids.reshape(_NW, _HPW // 128, 128),
        order.reshape(_NW, _HPW // 128, 128),
    )
    out = out.reshape(3, _BATCH * _DIM, 1)
    return out[0], out[1], out[2]


# speedup vs baseline: 1.0115x; 1.0115x over previous
import functools

import jax
import jax.numpy as jnp
from jax import lax
from jax.experimental import pallas as pl
from jax.experimental.pallas import tpu as pltpu
from jax.experimental.pallas import tpu_sc as plsc

_VOCAB = 1000000
_DIM = 64
_BATCH = 16384
_TOTAL = 3 * _BATCH

_NC = 2
_NS = 16
_NW = _NC * _NS
_HPW = _TOTAL // _NW

_LBLK = 768
_NFULL = _VOCAB // _LBLK
_TAIL = _VOCAB - _NFULL * _LBLK

_RING = 16

_mesh = plsc.VectorSubcoreMesh(core_axis_name="c", subcore_axis_name="s")


@functools.partial(
    pl.kernel,
    out_type=jax.ShapeDtypeStruct((_TOTAL * _DIM,), jnp.float32),
    mesh=_mesh,
    compiler_params=pltpu.CompilerParams(
        use_tc_tiling_on_sc=True, needs_layout_passes=False
    ),
    scratch_types=[
        pltpu.VMEM((_HPW // 128, 128), jnp.int32),
        pltpu.VMEM((_HPW // 128, 128), jnp.int32),
        pltpu.VMEM((2, _DIM, _LBLK), jnp.float32),
        pltpu.VMEM((_RING, _DIM), jnp.float32),
        pltpu.SemaphoreType.DMA,
        pltpu.SemaphoreType.DMA,
    ],
)
def _gather_kernel(tableT, tail_pad, ids_hbm, pos_hbm, out_hbm, ids_v, pos_v,
                   bufs_v, ring_v, sem_out, sem_blk):
    wid = lax.axis_index("s") * _NC + lax.axis_index("c")
    pltpu.sync_copy(ids_hbm.at[wid], ids_v)
    pltpu.sync_copy(pos_hbm.at[wid], pos_v)

    first_id = ids_v[0, pl.ds(0, 16)][0]
    lane = lax.iota(jnp.int32, 16)

    def drain_one_row():
        pltpu.make_async_copy(
            out_hbm.at[pl.ds(0, _DIM)], ring_v.at[0], sem_out
        ).wait()

    def wait_block():
        pltpu.make_async_copy(
            tableT.at[:, pl.ds(0, _LBLK)], bufs_v.at[0], sem_blk
        ).wait()

    def prefetch_block(pred, b, buf_slot):
        @pl.when(jnp.logical_and(pred, b != _NFULL))
        def _():
            pltpu.async_copy(
                tableT.at[:, pl.ds(b * _LBLK, _LBLK)],
                bufs_v.at[buf_slot], sem_blk,
            )

        @pl.when(jnp.logical_and(pred, b == _NFULL))
        def _():
            pltpu.async_copy(tail_pad, bufs_v.at[buf_slot], sem_blk)

    def process_group(g, carry):
        cur_blk, parity, pid = carry
        row = jnp.full((16,), g // 8, jnp.int32)
        colg = (lax.rem(g, jnp.int32(8)) * 16) + lane
        ids16 = plsc.load_gather(ids_v, [row, colg])
        pos16 = plsc.load_gather(pos_v, [row, colg])

        for j in range(16):
            nid = ids16[j]
            npos = pos16[j]
            blk = nid // _LBLK
            switch = blk != cur_blk
            hit_pf = jnp.logical_and(switch, pid == blk)
            gap = jnp.logical_and(switch, pid != blk)

            @pl.when(switch)
            def _():
                wait_block()

            @pl.when(jnp.logical_and(gap, blk != _NFULL))
            def _():
                pltpu.sync_copy(
                    tableT.at[:, pl.ds(blk * _LBLK, _LBLK)],
                    bufs_v.at[parity],
                )

            @pl.when(jnp.logical_and(gap, blk == _NFULL))
            def _():
                pltpu.sync_copy(tail_pad, bufs_v.at[parity])

            parity = jnp.where(hit_pf, 1 - parity, parity)
            cur_blk = jnp.where(switch, blk, cur_blk)
            tp = jnp.minimum(blk + 1, _NFULL)
            prefetch_block(switch, tp, 1 - parity)
            pid = jnp.where(switch, tp, pid)

            c = nid - blk * _LBLK
            slot = j

            drain_one_row()

            par16 = jnp.full((16,), parity, jnp.int32)
            col = jnp.full((16,), c, jnp.int32)
            for q in range(_DIM // 16):
                v = plsc.load_gather(bufs_v, [par16, lane + (16 * q), col])
                ring_v[slot, pl.ds(16 * q, 16)] = v
            pltpu.async_copy(
                ring_v.at[slot], out_hbm.at[pl.ds(npos * _DIM, _DIM)],
                sem_out,
            )

        return (cur_blk, parity, pid)

    for s in range(_RING):
        pltpu.async_copy(out_hbm.at[pl.ds(0, _DIM)], ring_v.at[s], sem_out)

    b0 = first_id // _LBLK

    @pl.when(b0 != _NFULL)
    def _():
        pltpu.sync_copy(tableT.at[:, pl.ds(b0 * _LBLK, _LBLK)], bufs_v.at[0])

    @pl.when(b0 == _NFULL)
    def _():
        pltpu.sync_copy(tail_pad, bufs_v.at[0])

    tp0 = jnp.minimum(b0 + 1, _NFULL)
    prefetch_block(jnp.bool_(True), tp0, 1)

    carry0 = (b0, jnp.int32(0), tp0)
    lax.fori_loop(0, _HPW // 16, process_group, carry0)

    wait_block()
    for _ in range(_RING):
        drain_one_row()


def kernel(anchor, pos, neg, embedding_table):
    tableT = embedding_table.T
    tail_pad = jnp.zeros((_DIM, _LBLK), jnp.float32)
    tail_pad = tail_pad.at[:, :_TAIL].set(embedding_table[_NFULL * _LBLK:, :].T)
    idx = jnp.concatenate([anchor, pos, neg]).astype(jnp.int32)
    sorted_ids, order = lax.sort_key_val(
        idx, lax.iota(jnp.int32, _TOTAL)
    )
    out = _gather_kernel(
        tableT,
        tail_pad,
        sorted_ids.reshape(_NW, _HPW // 128, 128),
        order.reshape(_NW, _HPW // 128, 128),
    )
    out = out.reshape(3, _BATCH * _DIM, 1)
    return out[0], out[1], out[2]
